# baseline (speedup 1.0000x reference)
import jax
import jax.numpy as jnp
from jax.experimental import pallas as pl
from jax.experimental.pallas import tpu as pltpu

NUM_CANVASES = 3
B, C, H, W = 16, 3, 512, 512

_ROWS = B * C * H
_N_CHUNKS = 24
_CHUNK_ROWS = _ROWS // _N_CHUNKS
_N_BUF = 8
_LAG = 4


def _copy_kernel(src_hbm, dst_hbm, bufs, in_sems, out_sems):
    def in_copy(j):
        b = j % _N_BUF
        return pltpu.make_async_copy(
            src_hbm.at[pl.ds(j * _CHUNK_ROWS, _CHUNK_ROWS)], bufs.at[b], in_sems.at[b]
        )

    def out_copy(j):
        b = j % _N_BUF
        return pltpu.make_async_copy(
            bufs.at[b], dst_hbm.at[pl.ds(j * _CHUNK_ROWS, _CHUNK_ROWS)], out_sems.at[b]
        )

    for j in range(_N_BUF):
        in_copy(j).start()
    for i in range(_N_CHUNKS):
        in_copy(i).wait()
        out_copy(i).start()
        if i >= _LAG:
            j = i - _LAG + _N_BUF
            if j < _N_CHUNKS:
                out_copy(i - _LAG).wait()
                in_copy(j).start()
    for i in range(_N_CHUNKS - _N_BUF, _N_CHUNKS):
        out_copy(i).wait()


def kernel(img_batch, canvases):
    del canvases
    flat = img_batch.reshape(_ROWS, W)
    out = pl.pallas_call(
        _copy_kernel,
        in_specs=[pl.BlockSpec(memory_space=pl.ANY)],
        out_specs=pl.BlockSpec(memory_space=pl.ANY),
        out_shape=jax.ShapeDtypeStruct((_ROWS, W), jnp.float32),
        scratch_shapes=[
            pltpu.VMEM((_N_BUF, _CHUNK_ROWS, W), jnp.float32),
            pltpu.SemaphoreType.DMA((_N_BUF,)),
            pltpu.SemaphoreType.DMA((_N_BUF,)),
        ],
    )(flat)
    return out.reshape(B, C, H, W)


# speedup vs baseline: 1.0242x; 1.0242x over previous
import jax
import jax.numpy as jnp
from jax.experimental import pallas as pl
from jax.experimental.pallas import tpu as pltpu

NUM_CANVASES = 3
B, C, H, W = 16, 3, 512, 512

_ROWS = B * C * H
_BLOCK_ROWS = 6144


def _copy_kernel(src_ref, dst_ref):
    dst_ref[...] = src_ref[...]


def kernel(img_batch, canvases):
    del canvases
    flat = img_batch.reshape(_ROWS, W)
    out = pl.pallas_call(
        _copy_kernel,
        grid=(_ROWS // _BLOCK_ROWS,),
        in_specs=[pl.BlockSpec((_BLOCK_ROWS, W), lambda i: (i, 0))],
        out_specs=pl.BlockSpec((_BLOCK_ROWS, W), lambda i: (i, 0)),
        out_shape=jax.ShapeDtypeStruct((_ROWS, W), jnp.float32),
        compiler_params=pltpu.CompilerParams(
            dimension_semantics=("parallel",),
        ),
    )(flat)
    return out.reshape(B, C, H, W)
